# baseline (speedup 1.0000x reference)
import functools

import jax
import jax.numpy as jnp
from jax.experimental import pallas as pl
from jax.experimental.pallas import tpu as pltpu

_HEADS = 16
_HID = 256


def _mm_kernel(nk, x_ref, w_ref, o_ref):
    k = pl.program_id(2)

    @pl.when(k == 0)
    def _init():
        o_ref[...] = jnp.zeros_like(o_ref)

    o_ref[...] += jnp.dot(x_ref[...], w_ref[...],
                          preferred_element_type=jnp.float32)


def _matmul(x, w, bm, bn, bk):
    m, kd = x.shape
    n = w.shape[1]
    mp = ((m + bm - 1) // bm) * bm
    if mp != m:
        x = jnp.pad(x, ((0, mp - m), (0, 0)))
    nm, nn, nk = mp // bm, n // bn, kd // bk
    out = pl.pallas_call(
        functools.partial(_mm_kernel, nk),
        grid=(nm, nn, nk),
        in_specs=[
            pl.BlockSpec((bm, bk), lambda i, j, k: (i, k)),
            pl.BlockSpec((bk, bn), lambda i, j, k: (k, j)),
        ],
        out_specs=pl.BlockSpec((bm, bn), lambda i, j, k: (i, j)),
        out_shape=jax.ShapeDtypeStruct((mp, n), jnp.float32),
        compiler_params=pltpu.CompilerParams(
            dimension_semantics=("parallel", "parallel", "arbitrary")),
    )(x, w)
    return out[:m]


def _edge_softmax(e, dst, n):
    e = jax.nn.leaky_relu(e, negative_slope=0.2)
    e_max = jax.ops.segment_max(e, dst, num_segments=n)
    ex = jnp.exp(e - e_max[dst])
    denom = jax.ops.segment_sum(ex, dst, num_segments=n)
    return ex / (denom[dst] + 1e-16)


def kernel(x, edge_index, W1, att_src1, att_dst1, b1, W2, att_src2,
           att_dst2, b2):
    n = x.shape[0]
    loop = jnp.arange(n, dtype=edge_index.dtype)
    src = jnp.concatenate([edge_index[0], loop])
    dst = jnp.concatenate([edge_index[1], loop])

    h = _matmul(x, W1, 512, 512, 256)
    h3 = h.reshape(n, _HEADS, _HID)
    a_src = jnp.einsum('nhc,hc->nh', h3, att_src1)
    a_dst = jnp.einsum('nhc,hc->nh', h3, att_dst1)
    alpha1 = _edge_softmax(a_src[src] + a_dst[dst], dst, n)
    msg = h3[src] * alpha1[:, :, None]
    out1 = jax.ops.segment_sum(msg, dst, num_segments=n)
    h1 = jax.nn.elu(out1.reshape(n, _HEADS * _HID) + b1)

    g = _matmul(h1, W2, 512, 256, 512)
    a_src2 = (g * att_src2[0][None, :]).sum(axis=-1, keepdims=True)
    a_dst2 = (g * att_dst2[0][None, :]).sum(axis=-1, keepdims=True)
    alpha2 = _edge_softmax(a_src2[src] + a_dst2[dst], dst, n)
    msg2 = g[src] * alpha2
    out2 = jax.ops.segment_sum(msg2, dst, num_segments=n)
    h2 = jax.nn.elu(out2 + b2)

    return (h2, alpha1, alpha2)
